# Initial kernel scaffold; baseline (speedup 1.0000x reference)
import jax
import jax.numpy as jnp
from jax.experimental import pallas as pl


def kernel(x_upd_core, edge_index_core, edge_attr_core, Z_core, Z_block, node2graph_core, Wn, bn, We, be, Wg, bg, Wm0, bm0, Wu0, bu0, Wm1, bm1, Wu1, bu1, W1, b1, W2, b2):
    raise NotImplementedError("write your pallas kernel here")


# speedup vs baseline: 3.9737x; 3.9737x over previous
import functools

import jax
import jax.numpy as jnp
from jax import lax
from jax.experimental import pallas as pl
from jax.experimental.pallas import tpu as pltpu
from jax.experimental.pallas import tpu_sc as plsc

N = 10000
E = 640000
G = 256
H = 64
DE = 16

NC = 2
NS = 16
NW = NC * NS
EPT = E // NW
C = 80
NCHUNK = EPT // C
RPT = 624
RPT_LAST = N - (NS - 1) * RPT

_F32 = jnp.float32


def _dot(a, b):
  return jnp.dot(a, b, preferred_element_type=_F32)


def _dotx(a, b):
  return jnp.dot(a, b, preferred_element_type=_F32,
                 precision=lax.Precision.HIGHEST)


@functools.cache
def _edge_sc_call():
  mesh = plsc.VectorSubcoreMesh(
      core_axis_name="c", subcore_axis_name="s", num_cores=NC, num_subcores=NS)

  @functools.partial(
      pl.kernel,
      out_type=jax.ShapeDtypeStruct((2 * N, 2 * H), _F32),
      mesh=mesh,
      scratch_types=[
          pltpu.VMEM((C,), jnp.int32),
          pltpu.VMEM((C,), jnp.int32),
          pltpu.VMEM((C, 2 * H), _F32),
          pltpu.VMEM((C, 2 * H), _F32),
          pltpu.VMEM((C, H), _F32),
          pltpu.VMEM((C, 2 * H), _F32),
          pltpu.VMEM_SHARED((N, 2 * H), _F32),
          pltpu.SemaphoreType.DMA,
      ],
  )
  def _edge_sc(t_hbm, ec_hbm, src_hbm, dst_hbm, zero_hbm, out_hbm,
               srcv, dstv, rowa, rowb, ecv, msg, acc, sem_g):
    c = lax.axis_index("c")
    s = lax.axis_index("s")
    base = (c * NS + s) * EPT

    @pl.when(s < NS - 1)
    def _():
      rows = pl.ds(s * RPT, RPT)
      pltpu.sync_copy(zero_hbm.at[pl.ds(0, RPT)], acc.at[rows])

    @pl.when(s == NS - 1)
    def _():
      rows = pl.ds((NS - 1) * RPT, RPT_LAST)
      pltpu.sync_copy(zero_hbm, acc.at[rows])

    plsc.subcore_barrier()

    zeros16 = jnp.zeros((16,), _F32)

    @pl.loop(0, C)
    def _zrows(r):
      for cc in range(H // 16):
        msg[r, pl.ds(H + cc * 16, 16)] = zeros16

    @pl.loop(0, NCHUNK)
    def _chunks(k):
      off = base + k * C
      pltpu.sync_copy(src_hbm.at[pl.ds(off, C)], srcv)
      pltpu.sync_copy(dst_hbm.at[pl.ds(off, C)], dstv)
      pltpu.sync_copy(ec_hbm.at[pl.ds(off, C)], ecv)
      pltpu.async_copy(t_hbm.at[srcv], rowa, sem_g).wait()
      pltpu.async_copy(t_hbm.at[dstv], rowb, sem_g).wait()

      @pl.loop(0, C)
      def _rows(r):
        for cc in range(H // 16):
          sl = pl.ds(cc * 16, 16)
          slb = pl.ds(H + cc * 16, 16)
          msg[r, sl] = jnp.maximum(rowa[r, sl] + rowb[r, slb] + ecv[r, sl], 0.0)

      pltpu.sync_copy(msg, acc.at[dstv], add=True)

    plsc.subcore_barrier()

    @pl.when(s < NS - 1)
    def _():
      pltpu.sync_copy(acc.at[pl.ds(s * RPT, RPT)],
                      out_hbm.at[pl.ds(c * N + s * RPT, RPT)])

    @pl.when(s == NS - 1)
    def _():
      pltpu.sync_copy(acc.at[pl.ds((NS - 1) * RPT, RPT_LAST)],
                      out_hbm.at[pl.ds(c * N + (NS - 1) * RPT, RPT_LAST)])

  return _edge_sc


EB = 3200


def _ec_body(ea, we, be_, w0, b0_, w1, b1_, o0, o1):
  e = jnp.maximum(_dot(ea[...], we[...]) + be_[...], 0.0)
  o0[...] = _dot(e, w0[...]) + b0_[...]
  o1[...] = _dot(e, w1[...]) + b1_[...]


_ec_call = pl.pallas_call(
    _ec_body,
    grid=(E // EB,),
    in_specs=[
        pl.BlockSpec((EB, DE), lambda i: (i, 0)),
        pl.BlockSpec((DE, H), lambda i: (0, 0)),
        pl.BlockSpec((1, H), lambda i: (0, 0)),
        pl.BlockSpec((H, H), lambda i: (0, 0)),
        pl.BlockSpec((1, H), lambda i: (0, 0)),
        pl.BlockSpec((H, H), lambda i: (0, 0)),
        pl.BlockSpec((1, H), lambda i: (0, 0)),
    ],
    out_specs=[
        pl.BlockSpec((EB, H), lambda i: (i, 0)),
        pl.BlockSpec((EB, H), lambda i: (i, 0)),
    ],
    out_shape=[jax.ShapeDtypeStruct((E, H), _F32)] * 2,
)


def _onehot(ids):
  return (ids == lax.broadcasted_iota(jnp.int32, (1, G), 1)).astype(_F32)


def _prep_body(x, n2g, zc, zb, wn, bn_, wg, bg_, wab, h0, t0):
  h = jnp.maximum(_dot(x[...], wn[...]) + bn_[...], 0.0)
  zcat = jnp.concatenate([zc[...], zb[...]], axis=1)
  g = jnp.maximum(_dot(zcat, wg[...]) + bg_[...], 0.0)
  h = h + _dotx(_onehot(n2g[...]), g)
  h0[...] = h
  t0[...] = _dot(h, wab[...])


_prep_call = pl.pallas_call(
    _prep_body,
    out_shape=[jax.ShapeDtypeStruct((N, H), _F32),
               jax.ShapeDtypeStruct((N, 2 * H), _F32)],
)


def _upd_body(h, agg, wuh, wua, bu_, wab, h1, t1):
  aggs = agg[0:N, 0:H] + agg[N:2 * N, 0:H]
  hh = h[...]
  hn = jnp.maximum(_dot(hh, wuh[...]) + _dot(aggs, wua[...]) + bu_[...],
                   0.0) + hh
  h1[...] = hn
  t1[...] = _dot(hn, wab[...])


_upd_call = pl.pallas_call(
    _upd_body,
    out_shape=[jax.ShapeDtypeStruct((N, H), _F32),
               jax.ShapeDtypeStruct((N, 2 * H), _F32)],
)


def _head_body(h, agg, wuh, wua, bu_, n2g, w1, b1_, w2, b2_, p_out):
  aggs = agg[0:N, 0:H] + agg[N:2 * N, 0:H]
  hh = h[...]
  hn = jnp.maximum(_dot(hh, wuh[...]) + _dot(aggs, wua[...]) + bu_[...],
                   0.0) + hh
  t = jnp.maximum(_dot(hn, w1[...]) + b1_[...], 0.0)
  logit = _dot(t, w2[...]) + b2_[...]
  ids = n2g[...]
  oh_b = ids == lax.broadcasted_iota(jnp.int32, (1, G), 1)
  oh = oh_b.astype(_F32)
  masked = jnp.where(oh_b, logit, -jnp.inf)
  segmax = jnp.max(masked, axis=0, keepdims=True)
  segmax = jnp.where(jnp.isfinite(segmax), segmax, 0.0)
  gmax = lax.dot_general(oh, segmax, (((1,), (1,)), ((), ())),
                         preferred_element_type=_F32,
                         precision=lax.Precision.HIGHEST)
  z = jnp.exp(logit - gmax)
  segsum = lax.dot_general(oh, z, (((0,), (0,)), ((), ())),
                           preferred_element_type=_F32,
                           precision=lax.Precision.HIGHEST)
  p_out[...] = z / _dotx(oh, segsum)


_head_call = pl.pallas_call(
    _head_body,
    out_shape=jax.ShapeDtypeStruct((N, 1), _F32),
)


def kernel(x_upd_core, edge_index_core, edge_attr_core, Z_core, Z_block,
           node2graph_core, Wn, bn, We, be, Wg, bg, Wm0, bm0, Wu0, bu0,
           Wm1, bm1, Wu1, bu1, W1, b1, W2, b2):
  src = edge_index_core[0].astype(jnp.int32)
  dst = edge_index_core[1].astype(jnp.int32)
  n2g = node2graph_core.astype(jnp.int32).reshape(N, 1)
  zero_rows = jnp.zeros((RPT_LAST, 2 * H), _F32)

  r2 = lambda v: v.reshape(1, -1)

  wab0 = jnp.concatenate([Wm0[:H], Wm0[H:2 * H]], axis=1)
  wab1 = jnp.concatenate([Wm1[:H], Wm1[H:2 * H]], axis=1)

  ec0, ec1 = _ec_call(edge_attr_core, We, r2(be),
                      Wm0[2 * H:], r2(bm0), Wm1[2 * H:], r2(bm1))

  h0, t0 = _prep_call(x_upd_core, n2g, Z_core, Z_block, Wn, r2(bn),
                      Wg, r2(bg), wab0)

  agg0 = _edge_sc_call()(t0, ec0, src, dst, zero_rows)

  h1, t1 = _upd_call(h0, agg0, Wu0[:H], Wu0[H:], r2(bu0), wab1)

  agg1 = _edge_sc_call()(t1, ec1, src, dst, zero_rows)

  p = _head_call(h1, agg1, Wu1[:H], Wu1[H:], r2(bu1), n2g,
                 W1, r2(b1), W2, r2(b2))
  return p.reshape(N)


# speedup vs baseline: 5.6950x; 1.4332x over previous
import functools

import jax
import jax.numpy as jnp
from jax import lax
from jax.experimental import pallas as pl
from jax.experimental.pallas import tpu as pltpu
from jax.experimental.pallas import tpu_sc as plsc

N = 10000
E = 640000
G = 256
H = 64
DE = 16

NC = 2
NS = 16
NW = NC * NS
EPT = E // NW
C = 40
NCHUNK = EPT // C
RPT = 624
RPT_LAST = N - (NS - 1) * RPT

_F32 = jnp.float32


def _dot(a, b):
  return jnp.dot(a, b, preferred_element_type=_F32)


def _dotx(a, b):
  return jnp.dot(a, b, preferred_element_type=_F32,
                 precision=lax.Precision.HIGHEST)


@functools.cache
def _edge_sc_call():
  mesh = plsc.VectorSubcoreMesh(
      core_axis_name="c", subcore_axis_name="s", num_cores=NC, num_subcores=NS)

  @functools.partial(
      pl.kernel,
      out_type=jax.ShapeDtypeStruct((2 * N, 2 * H), _F32),
      mesh=mesh,
      scratch_types=[
          pltpu.VMEM((C,), jnp.int32),
          pltpu.VMEM((C,), jnp.int32),
          pltpu.VMEM((C,), jnp.int32),
          pltpu.VMEM((C,), jnp.int32),
          pltpu.VMEM((C, 2 * H), _F32),
          pltpu.VMEM((C, 2 * H), _F32),
          pltpu.VMEM((C, 2 * H), _F32),
          pltpu.VMEM((C, 2 * H), _F32),
          pltpu.VMEM((C, H), _F32),
          pltpu.VMEM((C, H), _F32),
          pltpu.VMEM_SHARED((N, 2 * H), _F32),
          pltpu.SemaphoreType.DMA,
          pltpu.SemaphoreType.DMA,
          pltpu.SemaphoreType.DMA,
          pltpu.SemaphoreType.DMA,
          pltpu.SemaphoreType.DMA,
          pltpu.SemaphoreType.DMA,
      ],
  )
  def _edge_sc(t_hbm, ec_hbm, src_hbm, dst_hbm, zero_hbm, out_hbm,
               srcv0, srcv1, dstv0, dstv1, rowa0, rowa1, msg0, msg1,
               ecv0, ecv1, acc,
               semi0, semi1, semg0, semg1, sems0, sems1):
    srcv = (srcv0, srcv1)
    dstv = (dstv0, dstv1)
    rowa = (rowa0, rowa1)
    ecv = (ecv0, ecv1)
    msg = (msg0, msg1)
    semi = (semi0, semi1)
    semg = (semg0, semg1)
    sems = (sems0, sems1)
    c = lax.axis_index("c")
    s = lax.axis_index("s")
    base = (c * NS + s) * EPT

    @pl.when(s < NS - 1)
    def _():
      rows = pl.ds(s * RPT, RPT)
      pltpu.sync_copy(zero_hbm.at[pl.ds(0, RPT)], acc.at[rows])

    @pl.when(s == NS - 1)
    def _():
      rows = pl.ds((NS - 1) * RPT, RPT_LAST)
      pltpu.sync_copy(zero_hbm, acc.at[rows])

    plsc.subcore_barrier()

    @pl.loop(0, NCHUNK, step=2)
    def _chunks(g):
      din = []
      for j in range(2):
        off = base + (g + j) * C
        din.append((
            pltpu.async_copy(src_hbm.at[pl.ds(off, C)], srcv[j], semi[j]),
            pltpu.async_copy(dst_hbm.at[pl.ds(off, C)], dstv[j], semi[j]),
            pltpu.async_copy(ec_hbm.at[pl.ds(off, C)], ecv[j], semi[j]),
        ))
      dg = [None, None]
      for j in range(2):
        for d in din[j]:
          d.wait()
        dg[j] = (pltpu.async_copy(t_hbm.at[srcv[j]], rowa[j], semg[j]),
                 pltpu.async_copy(t_hbm.at[dstv[j]], msg[j], semg[j]))
      dsc = [None, None]
      for j in range(2):
        for d in dg[j]:
          d.wait()

        @pl.loop(0, C)
        def _rows(r):
          for cc in range(H // 16):
            sl = pl.ds(cc * 16, 16)
            slb = pl.ds(H + cc * 16, 16)
            msg[j][r, sl] = jnp.maximum(
                rowa[j][r, sl] + msg[j][r, slb] + ecv[j][r, sl], 0.0)

        dsc[j] = pltpu.async_copy(msg[j], acc.at[dstv[j]], sems[j], add=True)
      dsc[0].wait()
      dsc[1].wait()

    plsc.subcore_barrier()

    @pl.when(s < NS - 1)
    def _():
      pltpu.sync_copy(acc.at[pl.ds(s * RPT, RPT)],
                      out_hbm.at[pl.ds(c * N + s * RPT, RPT)])

    @pl.when(s == NS - 1)
    def _():
      pltpu.sync_copy(acc.at[pl.ds((NS - 1) * RPT, RPT_LAST)],
                      out_hbm.at[pl.ds(c * N + (NS - 1) * RPT, RPT_LAST)])

  return _edge_sc


EB = 3200


def _ec_body(ea, we, be_, w0, b0_, w1, b1_, o0, o1):
  e = jnp.maximum(_dot(ea[...], we[...]) + be_[...], 0.0)
  o0[...] = _dot(e, w0[...]) + b0_[...]
  o1[...] = _dot(e, w1[...]) + b1_[...]


_ec_call = pl.pallas_call(
    _ec_body,
    grid=(E // EB,),
    in_specs=[
        pl.BlockSpec((EB, DE), lambda i: (i, 0)),
        pl.BlockSpec((DE, H), lambda i: (0, 0)),
        pl.BlockSpec((1, H), lambda i: (0, 0)),
        pl.BlockSpec((H, H), lambda i: (0, 0)),
        pl.BlockSpec((1, H), lambda i: (0, 0)),
        pl.BlockSpec((H, H), lambda i: (0, 0)),
        pl.BlockSpec((1, H), lambda i: (0, 0)),
    ],
    out_specs=[
        pl.BlockSpec((EB, H), lambda i: (i, 0)),
        pl.BlockSpec((EB, H), lambda i: (i, 0)),
    ],
    out_shape=[jax.ShapeDtypeStruct((E, H), _F32)] * 2,
)


def _onehot(ids):
  return (ids == lax.broadcasted_iota(jnp.int32, (1, G), 1)).astype(_F32)


def _prep_body(x, n2g, zc, zb, wn, bn_, wg, bg_, wab, h0, t0):
  h = jnp.maximum(_dot(x[...], wn[...]) + bn_[...], 0.0)
  zcat = jnp.concatenate([zc[...], zb[...]], axis=1)
  g = jnp.maximum(_dot(zcat, wg[...]) + bg_[...], 0.0)
  h = h + _dotx(_onehot(n2g[...]), g)
  h0[...] = h
  t0[...] = _dot(h, wab[...])


_prep_call = pl.pallas_call(
    _prep_body,
    out_shape=[jax.ShapeDtypeStruct((N, H), _F32),
               jax.ShapeDtypeStruct((N, 2 * H), _F32)],
)


def _upd_body(h, agg, wuh, wua, bu_, wab, h1, t1):
  aggs = agg[0:N, 0:H] + agg[N:2 * N, 0:H]
  hh = h[...]
  hn = jnp.maximum(_dot(hh, wuh[...]) + _dot(aggs, wua[...]) + bu_[...],
                   0.0) + hh
  h1[...] = hn
  t1[...] = _dot(hn, wab[...])


_upd_call = pl.pallas_call(
    _upd_body,
    out_shape=[jax.ShapeDtypeStruct((N, H), _F32),
               jax.ShapeDtypeStruct((N, 2 * H), _F32)],
)


def _head_body(h, agg, wuh, wua, bu_, n2g, w1, b1_, w2, b2_, p_out):
  aggs = agg[0:N, 0:H] + agg[N:2 * N, 0:H]
  hh = h[...]
  hn = jnp.maximum(_dot(hh, wuh[...]) + _dot(aggs, wua[...]) + bu_[...],
                   0.0) + hh
  t = jnp.maximum(_dot(hn, w1[...]) + b1_[...], 0.0)
  logit = _dot(t, w2[...]) + b2_[...]
  ids = n2g[...]
  oh_b = ids == lax.broadcasted_iota(jnp.int32, (1, G), 1)
  oh = oh_b.astype(_F32)
  masked = jnp.where(oh_b, logit, -jnp.inf)
  segmax = jnp.max(masked, axis=0, keepdims=True)
  segmax = jnp.where(jnp.isfinite(segmax), segmax, 0.0)
  gmax = lax.dot_general(oh, segmax, (((1,), (1,)), ((), ())),
                         preferred_element_type=_F32,
                         precision=lax.Precision.HIGHEST)
  z = jnp.exp(logit - gmax)
  segsum = lax.dot_general(oh, z, (((0,), (0,)), ((), ())),
                           preferred_element_type=_F32,
                           precision=lax.Precision.HIGHEST)
  p_out[...] = z / _dotx(oh, segsum)


_head_call = pl.pallas_call(
    _head_body,
    out_shape=jax.ShapeDtypeStruct((N, 1), _F32),
)


def kernel(x_upd_core, edge_index_core, edge_attr_core, Z_core, Z_block,
           node2graph_core, Wn, bn, We, be, Wg, bg, Wm0, bm0, Wu0, bu0,
           Wm1, bm1, Wu1, bu1, W1, b1, W2, b2):
  src = edge_index_core[0].astype(jnp.int32)
  dst = edge_index_core[1].astype(jnp.int32)
  n2g = node2graph_core.astype(jnp.int32).reshape(N, 1)
  zero_rows = jnp.zeros((RPT_LAST, 2 * H), _F32)

  r2 = lambda v: v.reshape(1, -1)

  wab0 = jnp.concatenate([Wm0[:H], Wm0[H:2 * H]], axis=1)
  wab1 = jnp.concatenate([Wm1[:H], Wm1[H:2 * H]], axis=1)

  ec0, ec1 = _ec_call(edge_attr_core, We, r2(be),
                      Wm0[2 * H:], r2(bm0), Wm1[2 * H:], r2(bm1))

  h0, t0 = _prep_call(x_upd_core, n2g, Z_core, Z_block, Wn, r2(bn),
                      Wg, r2(bg), wab0)

  agg0 = _edge_sc_call()(t0, ec0, src, dst, zero_rows)

  h1, t1 = _upd_call(h0, agg0, Wu0[:H], Wu0[H:], r2(bu0), wab1)

  agg1 = _edge_sc_call()(t1, ec1, src, dst, zero_rows)

  p = _head_call(h1, agg1, Wu1[:H], Wu1[H:], r2(bu1), n2g,
                 W1, r2(b1), W2, r2(b2))
  return p.reshape(N)


# speedup vs baseline: 6.0321x; 1.0592x over previous
import functools

import jax
import jax.numpy as jnp
from jax import lax
from jax.experimental import pallas as pl
from jax.experimental.pallas import tpu as pltpu
from jax.experimental.pallas import tpu_sc as plsc

N = 10000
E = 640000
G = 256
H = 64
DE = 16

NC = 2
NS = 16
NW = NC * NS
EPT = E // NW
C = 40
NCHUNK = EPT // C
RPT = 624
RPT_LAST = N - (NS - 1) * RPT

_F32 = jnp.float32


def _dot(a, b):
  return jnp.dot(a, b, preferred_element_type=_F32)


def _dotx(a, b):
  return jnp.dot(a, b, preferred_element_type=_F32,
                 precision=lax.Precision.HIGHEST)


@functools.cache
def _edge_sc_call():
  mesh = plsc.VectorSubcoreMesh(
      core_axis_name="c", subcore_axis_name="s", num_cores=NC, num_subcores=NS)

  @functools.partial(
      pl.kernel,
      out_type=jax.ShapeDtypeStruct((2 * N, 2 * H), _F32),
      mesh=mesh,
      scratch_types=[
          pltpu.VMEM((C,), jnp.int32),
          pltpu.VMEM((C,), jnp.int32),
          pltpu.VMEM((C,), jnp.int32),
          pltpu.VMEM((C,), jnp.int32),
          pltpu.VMEM((C, 2 * H), _F32),
          pltpu.VMEM((C, 2 * H), _F32),
          pltpu.VMEM((C, 2 * H), _F32),
          pltpu.VMEM((C, 2 * H), _F32),
          pltpu.VMEM((C, H), _F32),
          pltpu.VMEM((C, H), _F32),
          pltpu.VMEM_SHARED((N, 2 * H), _F32),
          pltpu.SemaphoreType.DMA,
          pltpu.SemaphoreType.DMA,
          pltpu.SemaphoreType.DMA,
          pltpu.SemaphoreType.DMA,
          pltpu.SemaphoreType.DMA,
          pltpu.SemaphoreType.DMA,
      ],
  )
  def _edge_sc(t_hbm, ec_hbm, src_hbm, dst_hbm, zero_hbm, out_hbm,
               srcv0, srcv1, dstv0, dstv1, rowa0, rowa1, msg0, msg1,
               ecv0, ecv1, acc,
               semi0, semi1, semg0, semg1, sems0, sems1):
    srcv = (srcv0, srcv1)
    dstv = (dstv0, dstv1)
    rowa = (rowa0, rowa1)
    ecv = (ecv0, ecv1)
    msg = (msg0, msg1)
    semi = (semi0, semi1)
    semg = (semg0, semg1)
    sems = (sems0, sems1)
    c = lax.axis_index("c")
    s = lax.axis_index("s")
    base = (c * NS + s) * EPT

    @pl.when(s < NS - 1)
    def _():
      rows = pl.ds(s * RPT, RPT)
      pltpu.sync_copy(zero_hbm.at[pl.ds(0, RPT)], acc.at[rows])

    @pl.when(s == NS - 1)
    def _():
      rows = pl.ds((NS - 1) * RPT, RPT_LAST)
      pltpu.sync_copy(zero_hbm, acc.at[rows])

    plsc.subcore_barrier()

    def fire_in(k, j):
      off = base + k * C
      pltpu.async_copy(src_hbm.at[pl.ds(off, C)], srcv[j], semi[j])
      pltpu.async_copy(dst_hbm.at[pl.ds(off, C)], dstv[j], semi[j])
      pltpu.async_copy(ec_hbm.at[pl.ds(off, C)], ecv[j], semi[j])

    def wait_in(k, j):
      off = base + k * C
      pltpu.make_async_copy(src_hbm.at[pl.ds(off, C)], srcv[j], semi[j]).wait()
      pltpu.make_async_copy(dst_hbm.at[pl.ds(off, C)], dstv[j], semi[j]).wait()
      pltpu.make_async_copy(ec_hbm.at[pl.ds(off, C)], ecv[j], semi[j]).wait()

    def do_pair(g, fire_next):
      dg = [None, None]
      for j in range(2):
        wait_in(g + j, j)
        dg[j] = (pltpu.async_copy(t_hbm.at[srcv[j]], rowa[j], semg[j]),
                 pltpu.async_copy(t_hbm.at[dstv[j]], msg[j], semg[j]))
      dsc = [None, None]
      for j in range(2):
        for d in dg[j]:
          d.wait()

        @pl.loop(0, C)
        def _rows(r):
          for cc in range(H // 16):
            sl = pl.ds(cc * 16, 16)
            slb = pl.ds(H + cc * 16, 16)
            msg[j][r, sl] = jnp.maximum(
                rowa[j][r, sl] + msg[j][r, slb] + ecv[j][r, sl], 0.0)

        dsc[j] = pltpu.async_copy(msg[j], acc.at[dstv[j]], sems[j], add=True)
      for j in range(2):
        dsc[j].wait()
        if fire_next:
          fire_in(g + 2 + j, j)

    fire_in(0, 0)
    fire_in(1, 1)

    @pl.loop(0, NCHUNK - 2, step=2)
    def _chunks(g):
      do_pair(g, True)

    do_pair(NCHUNK - 2, False)

    plsc.subcore_barrier()

    @pl.when(s < NS - 1)
    def _():
      pltpu.sync_copy(acc.at[pl.ds(s * RPT, RPT)],
                      out_hbm.at[pl.ds(c * N + s * RPT, RPT)])

    @pl.when(s == NS - 1)
    def _():
      pltpu.sync_copy(acc.at[pl.ds((NS - 1) * RPT, RPT_LAST)],
                      out_hbm.at[pl.ds(c * N + (NS - 1) * RPT, RPT_LAST)])

  return _edge_sc


EB = 3200


def _ec_body(ea, we, be_, w0, b0_, w1, b1_, o0, o1):
  e = jnp.maximum(_dot(ea[...], we[...]) + be_[...], 0.0)
  o0[...] = _dot(e, w0[...]) + b0_[...]
  o1[...] = _dot(e, w1[...]) + b1_[...]


_ec_call = pl.pallas_call(
    _ec_body,
    grid=(E // EB,),
    in_specs=[
        pl.BlockSpec((EB, DE), lambda i: (i, 0)),
        pl.BlockSpec((DE, H), lambda i: (0, 0)),
        pl.BlockSpec((1, H), lambda i: (0, 0)),
        pl.BlockSpec((H, H), lambda i: (0, 0)),
        pl.BlockSpec((1, H), lambda i: (0, 0)),
        pl.BlockSpec((H, H), lambda i: (0, 0)),
        pl.BlockSpec((1, H), lambda i: (0, 0)),
    ],
    out_specs=[
        pl.BlockSpec((EB, H), lambda i: (i, 0)),
        pl.BlockSpec((EB, H), lambda i: (i, 0)),
    ],
    out_shape=[jax.ShapeDtypeStruct((E, H), _F32)] * 2,
)


def _onehot(ids):
  return (ids == lax.broadcasted_iota(jnp.int32, (1, G), 1)).astype(_F32)


def _prep_body(x, n2g, zc, zb, wn, bn_, wg, bg_, wab, h0, t0):
  h = jnp.maximum(_dot(x[...], wn[...]) + bn_[...], 0.0)
  zcat = jnp.concatenate([zc[...], zb[...]], axis=1)
  g = jnp.maximum(_dot(zcat, wg[...]) + bg_[...], 0.0)
  h = h + _dotx(_onehot(n2g[...]), g)
  h0[...] = h
  t0[...] = _dot(h, wab[...])


_prep_call = pl.pallas_call(
    _prep_body,
    out_shape=[jax.ShapeDtypeStruct((N, H), _F32),
               jax.ShapeDtypeStruct((N, 2 * H), _F32)],
)


def _upd_body(h, agg, wuh, wua, bu_, wab, h1, t1):
  aggs = agg[0:N, 0:H] + agg[N:2 * N, 0:H]
  hh = h[...]
  hn = jnp.maximum(_dot(hh, wuh[...]) + _dot(aggs, wua[...]) + bu_[...],
                   0.0) + hh
  h1[...] = hn
  t1[...] = _dot(hn, wab[...])


_upd_call = pl.pallas_call(
    _upd_body,
    out_shape=[jax.ShapeDtypeStruct((N, H), _F32),
               jax.ShapeDtypeStruct((N, 2 * H), _F32)],
)


def _head_body(h, agg, wuh, wua, bu_, n2g, w1, b1_, w2, b2_, p_out):
  aggs = agg[0:N, 0:H] + agg[N:2 * N, 0:H]
  hh = h[...]
  hn = jnp.maximum(_dot(hh, wuh[...]) + _dot(aggs, wua[...]) + bu_[...],
                   0.0) + hh
  t = jnp.maximum(_dot(hn, w1[...]) + b1_[...], 0.0)
  logit = _dot(t, w2[...]) + b2_[...]
  ids = n2g[...]
  oh_b = ids == lax.broadcasted_iota(jnp.int32, (1, G), 1)
  oh = oh_b.astype(_F32)
  masked = jnp.where(oh_b, logit, -jnp.inf)
  segmax = jnp.max(masked, axis=0, keepdims=True)
  segmax = jnp.where(jnp.isfinite(segmax), segmax, 0.0)
  gmax = lax.dot_general(oh, segmax, (((1,), (1,)), ((), ())),
                         preferred_element_type=_F32,
                         precision=lax.Precision.HIGHEST)
  z = jnp.exp(logit - gmax)
  segsum = lax.dot_general(oh, z, (((0,), (0,)), ((), ())),
                           preferred_element_type=_F32,
                           precision=lax.Precision.HIGHEST)
  p_out[...] = z / _dotx(oh, segsum)


_head_call = pl.pallas_call(
    _head_body,
    out_shape=jax.ShapeDtypeStruct((N, 1), _F32),
)


def kernel(x_upd_core, edge_index_core, edge_attr_core, Z_core, Z_block,
           node2graph_core, Wn, bn, We, be, Wg, bg, Wm0, bm0, Wu0, bu0,
           Wm1, bm1, Wu1, bu1, W1, b1, W2, b2):
  src = edge_index_core[0].astype(jnp.int32)
  dst = edge_index_core[1].astype(jnp.int32)
  n2g = node2graph_core.astype(jnp.int32).reshape(N, 1)
  zero_rows = jnp.zeros((RPT_LAST, 2 * H), _F32)

  r2 = lambda v: v.reshape(1, -1)

  wab0 = jnp.concatenate([Wm0[:H], Wm0[H:2 * H]], axis=1)
  wab1 = jnp.concatenate([Wm1[:H], Wm1[H:2 * H]], axis=1)

  ec0, ec1 = _ec_call(edge_attr_core, We, r2(be),
                      Wm0[2 * H:], r2(bm0), Wm1[2 * H:], r2(bm1))

  h0, t0 = _prep_call(x_upd_core, n2g, Z_core, Z_block, Wn, r2(bn),
                      Wg, r2(bg), wab0)

  agg0 = _edge_sc_call()(t0, ec0, src, dst, zero_rows)

  h1, t1 = _upd_call(h0, agg0, Wu0[:H], Wu0[H:], r2(bu0), wab1)

  agg1 = _edge_sc_call()(t1, ec1, src, dst, zero_rows)

  p = _head_call(h1, agg1, Wu1[:H], Wu1[H:], r2(bu1), n2g,
                 W1, r2(b1), W2, r2(b2))
  return p.reshape(N)
